# baseline (speedup 1.0000x reference)
from math import comb

import jax
import jax.numpy as jnp
import numpy as np
from jax import lax
from jax.experimental import pallas as pl
from jax.experimental.pallas import tpu as pltpu

_H, _W = 224, 224
_S = 15
_K = 4
_EPS = 1e-06
_BH = 32
_NB = _H // _BH


def _basis() -> jnp.ndarray:
    ts = jnp.linspace(0.0, 1.0, _S)
    i = np.arange(_K)
    coeff = np.array([comb(_K - 1, j) for j in range(_K)], dtype=np.float32)
    return (coeff[None, :] * (ts[:, None] ** i[None, :])
            * ((1.0 - ts[:, None]) ** (_K - 1 - i[None, :]))).astype(jnp.float32)


def _curve_kernel(s2_ref, ym_ref, xm_ref, slo_ref, scnt_ref, a2_ref, cb_ref,
                  w2_ref, out_ref, yf_s, xf_s, p2_s):
    b = pl.program_id(0)
    r = pl.program_id(1)

    @pl.when(jnp.logical_and(b == 0, r == 0))
    def _init():
        yf = lax.broadcasted_iota(jnp.int32, (_H, _W), 0).astype(jnp.float32)
        xf = lax.broadcasted_iota(jnp.int32, (_H, _W), 1).astype(jnp.float32)
        yf_s[...] = yf
        xf_s[...] = xf
        p2_s[...] = yf * yf + xf * xf

    cnt = scnt_ref[b, r]

    @pl.when(cnt == 0)
    def _skip():
        out_ref[...] = jnp.zeros((1, _BH, _W), jnp.float32)

    @pl.when(cnt > 0)
    def _compute():
        rows = pl.ds(r * _BH, _BH)
        yf = yf_s[rows, :]
        xf = xf_s[rows, :]
        p2 = p2_s[rows, :]

        m = None
        for s in range(_S):
            v = yf * ym_ref[b, s] + xf * xm_ref[b, s]
            d2 = (p2 + v) + s2_ref[b, s]
            m = d2 if m is None else jnp.minimum(m, d2)
        mh = jnp.maximum(m, 0.0) + 1e-12
        t = a2_ref[b] * jnp.log2(mh) + cb_ref[b]
        val = 1.0 - jnp.exp2(t)
        out_ref[0] = jnp.where(mh < w2_ref[b], val, 0.0)


@jax.jit
def kernel(inputs, widths, aa_factors):
    B = inputs.shape[0]
    kp = inputs * jnp.array([float(_H), float(_W)], dtype=jnp.float32)
    sp = jnp.einsum('sk,bkd->bsd', _basis(), kp)
    s2 = jnp.sum(sp * sp, axis=-1)

    def _rne_bf16(x):
        u = lax.bitcast_convert_type(x, jnp.uint32)
        u = u + jnp.uint32(0x7FFF) + ((u >> 16) & jnp.uint32(1))
        return lax.bitcast_convert_type(u & jnp.uint32(0xFFFF0000), jnp.float32)

    syq = _rne_bf16(sp[:, :, 0])
    sxq = _rne_bf16(sp[:, :, 1])
    ym = -2.0 * syq
    xm = -2.0 * sxq

    corr = s2 - (syq * syq + sxq * sxq)
    margin = jnp.sqrt(jnp.maximum(widths[:, None] ** 2 - corr, 0.0) + 2.0)
    r0 = (jnp.arange(_NB, dtype=jnp.float32) * _BH)[None, :, None]
    reach = ((syq[:, None, :] >= r0 - margin[:, None, :]) &
             (syq[:, None, :] <= r0 + (_BH - 1) + margin[:, None, :]))
    sidx = jnp.arange(_S, dtype=jnp.int32)
    slo = jnp.min(jnp.where(reach, sidx, _S), axis=-1).astype(jnp.int32)
    shi = jnp.max(jnp.where(reach, sidx, -1), axis=-1).astype(jnp.int32)
    scnt = jnp.where(shi >= slo, shi - slo + 1, 0).astype(jnp.int32)

    a2 = 0.5 * aa_factors
    cb = -aa_factors * jnp.log2(widths)
    w2 = widths * widths

    return pl.pallas_call(
        _curve_kernel,
        grid=(B, _NB),
        in_specs=[pl.BlockSpec(memory_space=pltpu.SMEM)] * 8,
        out_specs=pl.BlockSpec((1, _BH, _W), lambda b, r: (b, r, 0)),
        out_shape=jax.ShapeDtypeStruct((B, _H, _W), jnp.float32),
        scratch_shapes=[pltpu.VMEM((_H, _W), jnp.float32)] * 3,
    )(s2, ym, xm, slo, scnt, a2, cb, w2)


# speedup vs baseline: 2.9468x; 2.9468x over previous
from math import comb

import jax
import jax.numpy as jnp
import numpy as np
from jax import lax
from jax.experimental import pallas as pl
from jax.experimental.pallas import tpu as pltpu

_H, _W = 224, 224
_S = 15
_K = 4
_EPS = 1e-06
_BH = 32
_NB = _H // _BH


def _basis() -> jnp.ndarray:
    ts = jnp.linspace(0.0, 1.0, _S)
    i = np.arange(_K)
    coeff = np.array([comb(_K - 1, j) for j in range(_K)], dtype=np.float32)
    return (coeff[None, :] * (ts[:, None] ** i[None, :])
            * ((1.0 - ts[:, None]) ** (_K - 1 - i[None, :]))).astype(jnp.float32)


def _curve_kernel(s2_ref, ym_ref, xm_ref, scnt_ref, a2_ref, cb_ref,
                  w2_ref, out_ref, yf_s, xf_s, p2_s):
    b = pl.program_id(0)

    @pl.when(b == 0)
    def _init():
        yf = lax.broadcasted_iota(jnp.int32, (_H, _W), 0).astype(jnp.float32)
        xf = lax.broadcasted_iota(jnp.int32, (_H, _W), 1).astype(jnp.float32)
        yf_s[...] = yf
        xf_s[...] = xf
        p2_s[...] = yf * yf + xf * xf

    for r in range(_NB):
        rows = pl.ds(r * _BH, _BH)
        cnt = scnt_ref[b, r]

        @pl.when(cnt == 0)
        def _skip(rows=rows):
            out_ref[0, rows, :] = jnp.zeros((_BH, _W), jnp.float32)

        @pl.when(cnt > 0)
        def _compute(rows=rows):
            yf = yf_s[rows, :]
            xf = xf_s[rows, :]
            p2 = p2_s[rows, :]

            m = None
            for s in range(_S):
                v = yf * ym_ref[b, s] + xf * xm_ref[b, s]
                d2 = (p2 + v) + s2_ref[b, s]
                m = d2 if m is None else jnp.minimum(m, d2)
            mh = jnp.maximum(m, 0.0) + 1e-12
            t = a2_ref[b] * jnp.log2(mh) + cb_ref[b]
            val = 1.0 - jnp.exp2(t)
            out_ref[0, rows, :] = jnp.where(mh < w2_ref[b], val, 0.0)


@jax.jit
def kernel(inputs, widths, aa_factors):
    B = inputs.shape[0]
    kp = inputs * jnp.array([float(_H), float(_W)], dtype=jnp.float32)
    sp = jnp.einsum('sk,bkd->bsd', _basis(), kp)
    s2 = jnp.sum(sp * sp, axis=-1)

    def _rne_bf16(x):
        u = lax.bitcast_convert_type(x, jnp.uint32)
        u = u + jnp.uint32(0x7FFF) + ((u >> 16) & jnp.uint32(1))
        return lax.bitcast_convert_type(u & jnp.uint32(0xFFFF0000), jnp.float32)

    syq = _rne_bf16(sp[:, :, 0])
    sxq = _rne_bf16(sp[:, :, 1])
    ym = -2.0 * syq
    xm = -2.0 * sxq

    corr = s2 - (syq * syq + sxq * sxq)
    margin = jnp.sqrt(jnp.maximum(widths[:, None] ** 2 - corr, 0.0) + 2.0)
    r0 = (jnp.arange(_NB, dtype=jnp.float32) * _BH)[None, :, None]
    reach = ((syq[:, None, :] >= r0 - margin[:, None, :]) &
             (syq[:, None, :] <= r0 + (_BH - 1) + margin[:, None, :]))
    scnt = jnp.sum(reach, axis=-1).astype(jnp.int32)

    a2 = 0.5 * aa_factors
    cb = -aa_factors * jnp.log2(widths)
    w2 = widths * widths

    grid_spec = pltpu.PrefetchScalarGridSpec(
        num_scalar_prefetch=7,
        grid=(B,),
        in_specs=[],
        out_specs=pl.BlockSpec((1, _H, _W), lambda b, *_: (b, 0, 0)),
        scratch_shapes=[pltpu.VMEM((_H, _W), jnp.float32)] * 3,
    )
    return pl.pallas_call(
        _curve_kernel,
        grid_spec=grid_spec,
        out_shape=jax.ShapeDtypeStruct((B, _H, _W), jnp.float32),
    )(s2, ym, xm, scnt, a2, cb, w2)
